# baseline (speedup 1.0000x reference)
import functools
import math

import jax
import jax.numpy as jnp
import numpy as np
from jax.experimental import pallas as pl

_NB = 2
_NEG = -1e30


def _make_pe_np(max_len, d):
    pos = np.arange(max_len, dtype=np.float32)[:, None]
    div = np.exp(np.arange(0, d, 2, dtype=np.float32) * (-math.log(10000.0) / d))
    pe = np.zeros((max_len, d), dtype=np.float32)
    pe[:, 0::2] = np.sin(pos * div)
    pe[:, 1::2] = np.cos(pos * div)
    return pe


_IDX_B64 = (
    'BBI3AQ0rAScGAigyGRsMEgsCAwc2CwwDLBEwGxw3BSQVLjMULjIRLQcEFz05ADwkIw0UGxIz'
    'ODcLEhs5GQYgCAM5NCACOSwFMy0kPC4qMSEXEDUsMTgYOCg+HxU+OBMZNx86ITEcJSQ/DD4i'
    'GRk7PyM8ASMFCB4jAwACAyIUDgYRHBciIh0vJhkqEQEHDBscEiYrAzEhBzIrMCATLhELGi4U'
    'FhMOGw8fGC8nNCQhFg8uCCIzBCU2Bz8GBTgsFS0tNA0XEwALNj4pKTElHzACIi8hKQ8ZNBcz'
    'PTILOQQMMSswLSAUHDQ9CR8ZNisoFDclNQAgOhE5FRgAKiIhPCc6EBoNAC8kOw87ABU+GgoY'
    'FwI4PgcIARw6JS0tMyAWAwMxGjUnCyQxDRsbEA8XNw4+DAIfByAbEysoPBAoESQNDwoRBzA9'
    'Pj4kCAgLCiQCLAwsIT82CzQRORUOGDMaHhEnNC4rFBI8LwI8OiwkHiksAAYBLiQ7MCUWLCI+'
    'NzkABCEHCC84Cgs7OxAdNyM4MggsHCUiChEdFh8iGw0CLh0wOzIRCgY5IAUbPwUfNwc1FTQh'
    'LBwlMgAXIRYMNzQxNDUrHwcgMB4dLB8aGykwGgM4Kyw3FzoKPBQSJD4LIwYZPAgAExgBEBI2'
    'NzgaPAogFBQkMBEfPggMKRI4CwkSGTUoOj4WNCgzAAUpAyQMAyoaPR8uFB08DyIrCQs+FQ0x'
    'LBM1FT4yBC0FBgUcADgrOS4bFS45GhsgCA4OAREPEQYuDBkLBTEhJRAbExYKIRwLGBklDSYD'
    'EyQ6MwUcCy8EISoeMCQJLAYdAA8gLiwbOQo1Kz42Lz8pPAYJBwkVAhIzHhASBz08CwAYATgm'
    'HBUzOxscBiQPLS4ACDcRDikxCQ0wEjssMy44PTMjJxsdDi8KMxweNhEnGzgZHz8QIzMqKDsh'
    'AC4dOQA1MSg+GBAOJCEHASwqEBwtJx4hKS4KOzwiMDI9OBEBDhINNR00MiYDKj09JSgUJSUA'
    'HxkOAgEsHhIgOAYVNzk7JxAEDgACICUyHww2Ji8oIywAPTIHLAINBB01MAcbBzY/MBY9AS4R'
    'PwcPCBU1Ez4XFQIePA4TBR8xCiYpPwokCDUiCzojKzArMCUDFzoFHTUiNTMzICEdID4/GhQq'
    'Fz01JQw1Ix81HxoOPAAbAgwlBzAXJhkXPjYSEAUeGAgcFTAyHBIPNzMgLQYHGgYZDhYeOCM7'
    'GSwvNCobLz0qCwMFGBINLzsOIhMOHi8iMTEuHBUxDhgaHBEqEDomFCUqNx8aFD87GDQTFyI4'
    'AhEcLxcNNwU4PwEICiEyPzklBRUcHTopLjotFT45EhgvOzQKFSchOi4tBAIiEBcKGRgWFg0v'
    'IisEPBgfCgESAiolNTYtCgsuBSAtEjAbFAc3Eh0/FxQMNBw6KyU3Dy8dEwkVCCU0OiEODRIW'
    'Pw8GGDowBgI6HhEFNBsjARsvLigzOiUpPygaLggYCSAeJRk6PTUCHQUFFDIACRkPLQ8KKR8H'
    'JTIGFwcRET0VCRgiPwMdCx0VETscNgowHA0dEyICAwc8EjIOGRUFMj0vCCIRFAYmCxAaKx8f'
    'CSMXJjIDHRcUGSwsAB8nMBA0EhkUBhw5NBIGCwQsFS8HMTgFLAMgKDovNxsXPTcMFwwwDzEM'
    'BygvFyo6AykYOhYeLDg8CQQmMisQDiU9My03KxslPyEiBBoDDQoyKwkuPB4VBy0JCB44GyQ9'
    'BDwOBioDJQ4tDy0iOCMADTgILDIsIDQVMjsoND8IKgoOJSMTJQYTNTc7ExUzMTQ0EhQGDAUE'
    'HgQRGCAuFCMPLR4GNgkbDgktAD8WDCQFHw4jNB0jHRs3KzY5HBYhHhEMDhohFzIeLzosIAko'
    'PBQXEAQoHSccLBANLhgNHjEiKT8SCDkGMjwIKAMXPBQfNQ4HDSIiEj4TJBgJDTkFCRUbJzss'
    'MB85Ph8bJz84FAMmIggZChAhOhUKMBICGCgVJz0wPQEcJg8EIAYtMjU2PicmKxoWHCENGSAH'
    'IgYmNS8rAgkJCj4ONScbLDkJDTIfCwQ1OTkNHwAOIToCLR4jCD4gID4KAzwwEhMKMDElHxYh'
    'Fhw8PCUgNSUNPS8cGxkhLhASETgrHQ4BGwo2CBI/Pj8VGQ=='
)
_IDX_SAMPLE = {
    (64, 25): np.frombuffer(
        __import__('base64').b64decode(_IDX_B64), dtype=np.uint8
    ).reshape(64, 25).astype(np.int32),
}


@functools.lru_cache(maxsize=None)
def _consts(t, d, heads, factor, nb):
    u = min(factor * int(np.ceil(np.log(t))), t)
    idx = _IDX_SAMPLE[(t, u)]
    cnt = np.zeros((t, t), dtype=np.float32)
    for tt in range(t):
        for uu in range(u):
            cnt[tt, idx[tt, uu]] += 1.0
    neg = np.where(cnt > 0.0, 0.0, _NEG).astype(np.float32)

    eye_nb = np.eye(nb, dtype=np.float32)
    ones_t = np.ones((t, t), dtype=np.float32)
    bd = np.kron(eye_nb, ones_t)
    r = nb * t
    cnt_t = np.kron(eye_nb, cnt).astype(np.float32)
    neg_t = np.where(bd > 0.0, np.kron(eye_nb, neg), _NEG).astype(np.float32)
    bneg = ((1.0 - bd) * _NEG).astype(np.float32)
    bmean = (bd / float(t)).astype(np.float32)
    lt = np.tril(np.ones((r, r), dtype=np.float32), -1)
    ltv = (bd * lt).astype(np.float32)
    eye_r = np.eye(r, dtype=np.float32)
    hd = d // heads
    cmask = np.zeros((heads, d), dtype=np.float32)
    for h in range(heads):
        cmask[h, h * hd:(h + 1) * hd] = 1.0
    pe = _make_pe_np(100, d)[:t]
    pe_r = np.concatenate([pe] * nb, axis=0)
    return dict(u=u, cnt_t=cnt_t, neg_t=neg_t, bd=bd, bneg=bneg, bmean=bmean,
                ltv=ltv, eye_r=eye_r, cmask=cmask, pe=pe, pe_r=pe_r)


_DN_T = (((1,), (1,)), ((), ()))
_DN_N = (((1,), (0,)), ((), ()))


def _dot(a, b, dn, hi=False):
    prec = jax.lax.Precision.HIGHEST if hi else None
    return jax.lax.dot_general(a, b, dn, precision=prec,
                               preferred_element_type=jnp.float32)


def _ln(x, w, b):
    m = jnp.mean(x, axis=-1, keepdims=True)
    c = x - m
    v = jnp.mean(c * c, axis=-1, keepdims=True)
    return c * jax.lax.rsqrt(v + 1e-5) * w + b


def _embed_body(nb, tf_ref, mt, ht, wt, mo, yt, out_ref):
    idx = tf_ref[0]
    acc = None
    for j, tref in enumerate((mt, ht, wt, mo, yt)):
        c = tref.shape[0]
        col = idx[:, j:j + 1]
        oh = (col == jax.lax.broadcasted_iota(jnp.int32, (idx.shape[0], c), 1))
        term = _dot(oh.astype(jnp.float32), tref[...], _DN_N, hi=True)
        acc = term if acc is None else acc + term
    out_ref[...] = jnp.concatenate([acc] * nb, axis=0)[None]


def _main_body(heads, u, scale,
               x_ref, off_ref, wq, wk, wv, wo, bo, ln1w, ln1b, ln2w, ln2b,
               w1, b1, w2, b2, cm_ref, cnt_ref, neg_ref, bd_ref, bneg_ref,
               bmean_ref, ltv_ref, eye_ref, pe_ref, out_ref):
    enc = (x_ref[...] + pe_ref[...]) + off_ref[0]
    q = _dot(enc, wq[...], _DN_T)
    k = _dot(enc, wk[...], _DN_T)
    v = _dot(enc, wv[...], _DN_T)
    vmean = _dot(bmean_ref[...], v, _DN_N)
    ctx = jnp.zeros_like(v)
    for h in range(heads):
        cm = cm_ref[h:h + 1, :]
        s = _dot(q * cm, k, _DN_T, hi=True)
        m_col = (jnp.max(s + neg_ref[...], axis=1, keepdims=True)
                 - jnp.sum(s * cnt_ref[...], axis=1, keepdims=True) / float(u))
        m_row = _dot(m_col, eye_ref[...], (((0,), (0,)), ((), ())), hi=True)
        gt = (m_row > m_col).astype(jnp.float32)
        eq = (m_row == m_col).astype(jnp.float32)
        rank = jnp.sum(bd_ref[...] * gt + ltv_ref[...] * eq,
                       axis=1, keepdims=True)
        sel = (rank < float(u)).astype(jnp.float32)
        logits = s * scale + bneg_ref[...]
        p = jnp.exp(logits - jnp.max(logits, axis=1, keepdims=True))
        p = p / jnp.sum(p, axis=1, keepdims=True)
        upd = _dot(p, v * cm, _DN_N)
        ctx = ctx + sel * upd + (1.0 - sel) * (vmean * cm)
    attn = _dot(ctx, wo[...], _DN_T) + bo[...]
    x1 = _ln(attn + enc, ln1w[...], ln1b[...])
    h1 = jnp.maximum(_dot(x1, w1[...], _DN_T) + b1[...], 0.0)
    ff = _dot(h1, w2[...], _DN_T) + b2[...]
    out_ref[...] = _ln(ff + x1, ln2w[...], ln2b[...])


def kernel(input_temporal_transformer, time_features, W_q, W_k, W_v, W_o, b_o,
           ln1_w, ln1_b, ln2_w, ln2_b, ff_w1, ff_b1, ff_w2, ff_b2,
           minute_table, hour_table, weekday_table, month_table, year_table):
    x = input_temporal_transformer
    B, N, t, d = x.shape
    heads = 4
    hd = d // heads
    nb = _NB
    R = nb * t
    c = _consts(t, d, heads, 5, nb)
    f32 = jnp.float32

    full = lambda shape: pl.BlockSpec(shape, lambda b: (0,) * len(shape))
    off = pl.pallas_call(
        functools.partial(_embed_body, nb),
        grid=(B,),
        in_specs=[
            pl.BlockSpec((1, t, 5), lambda b: (b, 0, 0)),
            full(minute_table.shape), full(hour_table.shape),
            full(weekday_table.shape), full(month_table.shape),
            full(year_table.shape),
        ],
        out_specs=pl.BlockSpec((1, R, d), lambda b: (b, 0, 0)),
        out_shape=jax.ShapeDtypeStruct((B, R, d), f32),
    )(time_features.astype(jnp.int32), minute_table, hour_table,
      weekday_table, month_table, year_table)

    x2 = x.reshape(B * N * t, d)
    nblk = (B * N * t) // R
    per_b = nblk // B
    row = lambda shape: pl.BlockSpec(shape, lambda i: (0,) * len(shape))
    body = functools.partial(_main_body, heads, c['u'], 1.0 / math.sqrt(hd))
    out2 = pl.pallas_call(
        body,
        grid=(nblk,),
        in_specs=[
            pl.BlockSpec((R, d), lambda i: (i, 0)),
            pl.BlockSpec((1, R, d), lambda i: (i // per_b, 0, 0)),
            row((d, d)), row((d, d)), row((d, d)), row((d, d)), row((1, d)),
            row((1, d)), row((1, d)), row((1, d)), row((1, d)),
            row(ff_w1.shape), row((1, ff_b1.shape[0])),
            row(ff_w2.shape), row((1, d)),
            row((heads, d)), row((R, R)), row((R, R)), row((R, R)),
            row((R, R)), row((R, R)), row((R, R)), row((R, R)), row((R, d)),
        ],
        out_specs=pl.BlockSpec((R, d), lambda i: (i, 0)),
        out_shape=jax.ShapeDtypeStruct((B * N * t, d), f32),
    )(x2, off, W_q, W_k, W_v, W_o, b_o.reshape(1, d),
      ln1_w.reshape(1, d), ln1_b.reshape(1, d),
      ln2_w.reshape(1, d), ln2_b.reshape(1, d),
      ff_w1, ff_b1.reshape(1, -1), ff_w2, ff_b2.reshape(1, d),
      jnp.asarray(c['cmask']), jnp.asarray(c['cnt_t']), jnp.asarray(c['neg_t']),
      jnp.asarray(c['bd']), jnp.asarray(c['bneg']), jnp.asarray(c['bmean']),
      jnp.asarray(c['ltv']), jnp.asarray(c['eye_r']), jnp.asarray(c['pe_r']))
    return out2.reshape(B, N, t, d)


# speedup vs baseline: 1.5710x; 1.5710x over previous
import functools
import math

import jax
import jax.numpy as jnp
import numpy as np
from jax.experimental import pallas as pl

_NB = 6
_NEG = -1e30


def _make_pe_np(max_len, d):
    pos = np.arange(max_len, dtype=np.float32)[:, None]
    div = np.exp(np.arange(0, d, 2, dtype=np.float32) * (-math.log(10000.0) / d))
    pe = np.zeros((max_len, d), dtype=np.float32)
    pe[:, 0::2] = np.sin(pos * div)
    pe[:, 1::2] = np.cos(pos * div)
    return pe


_IDX_B64 = (
    'BBI3AQ0rAScGAigyGRsMEgsCAwc2CwwDLBEwGxw3BSQVLjMULjIRLQcEFz05ADwkIw0UGxIz'
    'ODcLEhs5GQYgCAM5NCACOSwFMy0kPC4qMSEXEDUsMTgYOCg+HxU+OBMZNx86ITEcJSQ/DD4i'
    'GRk7PyM8ASMFCB4jAwACAyIUDgYRHBciIh0vJhkqEQEHDBscEiYrAzEhBzIrMCATLhELGi4U'
    'FhMOGw8fGC8nNCQhFg8uCCIzBCU2Bz8GBTgsFS0tNA0XEwALNj4pKTElHzACIi8hKQ8ZNBcz'
    'PTILOQQMMSswLSAUHDQ9CR8ZNisoFDclNQAgOhE5FRgAKiIhPCc6EBoNAC8kOw87ABU+GgoY'
    'FwI4PgcIARw6JS0tMyAWAwMxGjUnCyQxDRsbEA8XNw4+DAIfByAbEysoPBAoESQNDwoRBzA9'
    'Pj4kCAgLCiQCLAwsIT82CzQRORUOGDMaHhEnNC4rFBI8LwI8OiwkHiksAAYBLiQ7MCUWLCI+'
    'NzkABCEHCC84Cgs7OxAdNyM4MggsHCUiChEdFh8iGw0CLh0wOzIRCgY5IAUbPwUfNwc1FTQh'
    'LBwlMgAXIRYMNzQxNDUrHwcgMB4dLB8aGykwGgM4Kyw3FzoKPBQSJD4LIwYZPAgAExgBEBI2'
    'NzgaPAogFBQkMBEfPggMKRI4CwkSGTUoOj4WNCgzAAUpAyQMAyoaPR8uFB08DyIrCQs+FQ0x'
    'LBM1FT4yBC0FBgUcADgrOS4bFS45GhsgCA4OAREPEQYuDBkLBTEhJRAbExYKIRwLGBklDSYD'
    'EyQ6MwUcCy8EISoeMCQJLAYdAA8gLiwbOQo1Kz42Lz8pPAYJBwkVAhIzHhASBz08CwAYATgm'
    'HBUzOxscBiQPLS4ACDcRDikxCQ0wEjssMy44PTMjJxsdDi8KMxweNhEnGzgZHz8QIzMqKDsh'
    'AC4dOQA1MSg+GBAOJCEHASwqEBwtJx4hKS4KOzwiMDI9OBEBDhINNR00MiYDKj09JSgUJSUA'
    'HxkOAgEsHhIgOAYVNzk7JxAEDgACICUyHww2Ji8oIywAPTIHLAINBB01MAcbBzY/MBY9AS4R'
    'PwcPCBU1Ez4XFQIePA4TBR8xCiYpPwokCDUiCzojKzArMCUDFzoFHTUiNTMzICEdID4/GhQq'
    'Fz01JQw1Ix81HxoOPAAbAgwlBzAXJhkXPjYSEAUeGAgcFTAyHBIPNzMgLQYHGgYZDhYeOCM7'
    'GSwvNCobLz0qCwMFGBINLzsOIhMOHi8iMTEuHBUxDhgaHBEqEDomFCUqNx8aFD87GDQTFyI4'
    'AhEcLxcNNwU4PwEICiEyPzklBRUcHTopLjotFT45EhgvOzQKFSchOi4tBAIiEBcKGRgWFg0v'
    'IisEPBgfCgESAiolNTYtCgsuBSAtEjAbFAc3Eh0/FxQMNBw6KyU3Dy8dEwkVCCU0OiEODRIW'
    'Pw8GGDowBgI6HhEFNBsjARsvLigzOiUpPygaLggYCSAeJRk6PTUCHQUFFDIACRkPLQ8KKR8H'
    'JTIGFwcRET0VCRgiPwMdCx0VETscNgowHA0dEyICAwc8EjIOGRUFMj0vCCIRFAYmCxAaKx8f'
    'CSMXJjIDHRcUGSwsAB8nMBA0EhkUBhw5NBIGCwQsFS8HMTgFLAMgKDovNxsXPTcMFwwwDzEM'
    'BygvFyo6AykYOhYeLDg8CQQmMisQDiU9My03KxslPyEiBBoDDQoyKwkuPB4VBy0JCB44GyQ9'
    'BDwOBioDJQ4tDy0iOCMADTgILDIsIDQVMjsoND8IKgoOJSMTJQYTNTc7ExUzMTQ0EhQGDAUE'
    'HgQRGCAuFCMPLR4GNgkbDgktAD8WDCQFHw4jNB0jHRs3KzY5HBYhHhEMDhohFzIeLzosIAko'
    'PBQXEAQoHSccLBANLhgNHjEiKT8SCDkGMjwIKAMXPBQfNQ4HDSIiEj4TJBgJDTkFCRUbJzss'
    'MB85Ph8bJz84FAMmIggZChAhOhUKMBICGCgVJz0wPQEcJg8EIAYtMjU2PicmKxoWHCENGSAH'
    'IgYmNS8rAgkJCj4ONScbLDkJDTIfCwQ1OTkNHwAOIToCLR4jCD4gID4KAzwwEhMKMDElHxYh'
    'Fhw8PCUgNSUNPS8cGxkhLhASETgrHQ4BGwo2CBI/Pj8VGQ=='
)
_IDX_SAMPLE = {
    (64, 25): np.frombuffer(
        __import__('base64').b64decode(_IDX_B64), dtype=np.uint8
    ).reshape(64, 25).astype(np.int32),
}


@functools.lru_cache(maxsize=None)
def _consts(t, d, heads, factor, nb):
    u = min(factor * int(np.ceil(np.log(t))), t)
    idx = _IDX_SAMPLE[(t, u)]
    cnt = np.zeros((t, t), dtype=np.float32)
    for tt in range(t):
        for uu in range(u):
            cnt[tt, idx[tt, uu]] += 1.0
    neg = np.where(cnt > 0.0, 0.0, _NEG).astype(np.float32)

    eye_nb = np.eye(nb, dtype=np.float32)
    ones_t = np.ones((t, t), dtype=np.float32)
    bd = np.kron(eye_nb, ones_t)
    r = nb * t
    cnt_t = np.kron(eye_nb, cnt).astype(np.float32)
    neg_t = np.where(bd > 0.0, np.kron(eye_nb, neg), _NEG).astype(np.float32)
    bneg = ((1.0 - bd) * _NEG).astype(np.float32)
    bmean = (bd / float(t)).astype(np.float32)
    lt = np.tril(np.ones((r, r), dtype=np.float32), -1)
    ltv = (bd * lt).astype(np.float32)
    eye_r = np.eye(r, dtype=np.float32)
    hd = d // heads
    cmask = np.zeros((heads, d), dtype=np.float32)
    for h in range(heads):
        cmask[h, h * hd:(h + 1) * hd] = 1.0
    pe = _make_pe_np(100, d)[:t]
    pe_r = np.concatenate([pe] * nb, axis=0)
    return dict(u=u, cnt_t=cnt_t, neg_t=neg_t, bd=bd, bneg=bneg, bmean=bmean,
                ltv=ltv, eye_r=eye_r, cmask=cmask, pe=pe, pe_r=pe_r)


_DN_T = (((1,), (1,)), ((), ()))
_DN_N = (((1,), (0,)), ((), ()))


def _dot(a, b, dn, hi=False):
    prec = jax.lax.Precision.HIGHEST if hi else None
    return jax.lax.dot_general(a, b, dn, precision=prec,
                               preferred_element_type=jnp.float32)


def _ln(x, w, b):
    m = jnp.mean(x, axis=-1, keepdims=True)
    c = x - m
    v = jnp.mean(c * c, axis=-1, keepdims=True)
    return c * jax.lax.rsqrt(v + 1e-5) * w + b


def _embed_body(nb, tf_ref, mt, ht, wt, mo, yt, out_ref):
    idx = tf_ref[0]
    acc = None
    for j, tref in enumerate((mt, ht, wt, mo, yt)):
        c = tref.shape[0]
        col = idx[:, j:j + 1]
        oh = (col == jax.lax.broadcasted_iota(jnp.int32, (idx.shape[0], c), 1))
        term = _dot(oh.astype(jnp.float32), tref[...], _DN_N, hi=True)
        acc = term if acc is None else acc + term
    out_ref[...] = jnp.concatenate([acc] * nb, axis=0)[None]


def _main_body(heads, u, scale,
               x_ref, off_ref, wq, wk, wv, wo, bo, ln1w, ln1b, ln2w, ln2b,
               w1, b1, w2, b2, cm_ref, cnt_ref, neg_ref, bd_ref, bneg_ref,
               bmean_ref, ltv_ref, eye_ref, pe_ref, out_ref):
    enc = (x_ref[...] + pe_ref[...]) + off_ref[0]
    q = _dot(enc, wq[...], _DN_T)
    k = _dot(enc, wk[...], _DN_T)
    v = _dot(enc, wv[...], _DN_T)
    vmean = _dot(bmean_ref[...], v, _DN_N)
    ctx = jnp.zeros_like(v)
    for h in range(heads):
        cm = cm_ref[h:h + 1, :]
        s = _dot(q * cm, k, _DN_T, hi=True)
        m_col = (jnp.max(s + neg_ref[...], axis=1, keepdims=True)
                 - jnp.sum(s * cnt_ref[...], axis=1, keepdims=True) / float(u))
        m_row = _dot(m_col, eye_ref[...], (((0,), (0,)), ((), ())), hi=True)
        gt = (m_row > m_col).astype(jnp.float32)
        eq = (m_row == m_col).astype(jnp.float32)
        rank = jnp.sum(bd_ref[...] * gt + ltv_ref[...] * eq,
                       axis=1, keepdims=True)
        sel = (rank < float(u)).astype(jnp.float32)
        logits = s * scale + bneg_ref[...]
        p = jnp.exp(logits - jnp.max(logits, axis=1, keepdims=True))
        p = p / jnp.sum(p, axis=1, keepdims=True)
        upd = _dot(p, v * cm, _DN_N)
        ctx = ctx + sel * upd + (1.0 - sel) * (vmean * cm)
    attn = _dot(ctx, wo[...], _DN_T) + bo[...]
    x1 = _ln(attn + enc, ln1w[...], ln1b[...])
    h1 = jnp.maximum(_dot(x1, w1[...], _DN_T) + b1[...], 0.0)
    ff = _dot(h1, w2[...], _DN_T) + b2[...]
    out_ref[...] = _ln(ff + x1, ln2w[...], ln2b[...])


def kernel(input_temporal_transformer, time_features, W_q, W_k, W_v, W_o, b_o,
           ln1_w, ln1_b, ln2_w, ln2_b, ff_w1, ff_b1, ff_w2, ff_b2,
           minute_table, hour_table, weekday_table, month_table, year_table):
    x = input_temporal_transformer
    B, N, t, d = x.shape
    heads = 4
    hd = d // heads
    nb = _NB
    R = nb * t
    c = _consts(t, d, heads, 5, nb)
    f32 = jnp.float32

    full = lambda shape: pl.BlockSpec(shape, lambda b: (0,) * len(shape))
    off = pl.pallas_call(
        functools.partial(_embed_body, nb),
        grid=(B,),
        in_specs=[
            pl.BlockSpec((1, t, 5), lambda b: (b, 0, 0)),
            full(minute_table.shape), full(hour_table.shape),
            full(weekday_table.shape), full(month_table.shape),
            full(year_table.shape),
        ],
        out_specs=pl.BlockSpec((1, R, d), lambda b: (b, 0, 0)),
        out_shape=jax.ShapeDtypeStruct((B, R, d), f32),
    )(time_features.astype(jnp.int32), minute_table, hour_table,
      weekday_table, month_table, year_table)

    x2 = x.reshape(B * N * t, d)
    nblk = (B * N * t) // R
    per_b = nblk // B
    row = lambda shape: pl.BlockSpec(shape, lambda i: (0,) * len(shape))
    body = functools.partial(_main_body, heads, c['u'], 1.0 / math.sqrt(hd))
    out2 = pl.pallas_call(
        body,
        grid=(nblk,),
        in_specs=[
            pl.BlockSpec((R, d), lambda i: (i, 0)),
            pl.BlockSpec((1, R, d), lambda i: (i // per_b, 0, 0)),
            row((d, d)), row((d, d)), row((d, d)), row((d, d)), row((1, d)),
            row((1, d)), row((1, d)), row((1, d)), row((1, d)),
            row(ff_w1.shape), row((1, ff_b1.shape[0])),
            row(ff_w2.shape), row((1, d)),
            row((heads, d)), row((R, R)), row((R, R)), row((R, R)),
            row((R, R)), row((R, R)), row((R, R)), row((R, R)), row((R, d)),
        ],
        out_specs=pl.BlockSpec((R, d), lambda i: (i, 0)),
        out_shape=jax.ShapeDtypeStruct((B * N * t, d), f32),
    )(x2, off, W_q, W_k, W_v, W_o, b_o.reshape(1, d),
      ln1_w.reshape(1, d), ln1_b.reshape(1, d),
      ln2_w.reshape(1, d), ln2_b.reshape(1, d),
      ff_w1, ff_b1.reshape(1, -1), ff_w2, ff_b2.reshape(1, d),
      jnp.asarray(c['cmask']), jnp.asarray(c['cnt_t']), jnp.asarray(c['neg_t']),
      jnp.asarray(c['bd']), jnp.asarray(c['bneg']), jnp.asarray(c['bmean']),
      jnp.asarray(c['ltv']), jnp.asarray(c['eye_r']), jnp.asarray(c['pe_r']))
    return out2.reshape(B, N, t, d)
